# baseline (speedup 1.0000x reference)
import jax
import jax.numpy as jnp
from jax.experimental import pallas as pl
from jax.experimental.pallas import tpu as pltpu

B, N, K = 131072, 1024, 64
BM = 512

_HI = jax.lax.Precision.HIGHEST


def _main_body(x_ref, wn_ref, a1_ref, a2_ref, proj_ref, v_ref):
    wn = wn_ref[...]
    b = jax.lax.dot_general(x_ref[...], wn, (((1,), (1,)), ((), ())),
                            preferred_element_type=jnp.float32)
    t = jnp.dot(b, a1_ref[...], precision=_HI,
                preferred_element_type=jnp.float32)
    v = jnp.dot(t, a2_ref[...], precision=_HI,
                preferred_element_type=jnp.float32)
    v_ref[...] = v
    proj_ref[...] = jnp.dot(v, wn, preferred_element_type=jnp.float32)


def kernel(x, w, r):
    w_new = r @ w
    wn = w_new / jnp.linalg.norm(w_new, axis=1, keepdims=True)
    phi = wn @ wn.T

    lu, _, perm = jax.lax.linalg.lu(phi)
    eye = jnp.eye(K, dtype=jnp.float32)
    l_fac = jnp.tril(lu, -1) + eye
    u_fac = jnp.triu(lu)
    l_inv = jax.scipy.linalg.solve_triangular(l_fac, eye, lower=True,
                                              unit_diagonal=True)
    u_inv = jax.scipy.linalg.solve_triangular(u_fac, eye, lower=False)
    p_mat = jax.nn.one_hot(perm, K, dtype=jnp.float32)
    a1 = jnp.dot(l_inv, p_mat, precision=_HI).T
    a2 = u_inv.T

    proj, v = pl.pallas_call(
        _main_body,
        out_shape=[
            jax.ShapeDtypeStruct((B, N), jnp.float32),
            jax.ShapeDtypeStruct((B, K), jnp.float32),
        ],
        grid=(B // BM,),
        in_specs=[
            pl.BlockSpec((BM, N), lambda i: (i, 0)),
            pl.BlockSpec((K, N), lambda i: (0, 0)),
            pl.BlockSpec((K, K), lambda i: (0, 0)),
            pl.BlockSpec((K, K), lambda i: (0, 0)),
        ],
        out_specs=[
            pl.BlockSpec((BM, N), lambda i: (i, 0)),
            pl.BlockSpec((BM, K), lambda i: (i, 0)),
        ],
        compiler_params=pltpu.CompilerParams(
            dimension_semantics=("parallel",),
        ),
        name="spaceopt_main",
    )(x, wn, a1, a2)
    return (proj, v)


# speedup vs baseline: 1.1943x; 1.1943x over previous
import jax
import jax.numpy as jnp
from jax.experimental import pallas as pl
from jax.experimental.pallas import tpu as pltpu

B, N, K = 131072, 1024, 64
BM = 1024

_HI = jax.lax.Precision.HIGHEST


def _split3(m):
    m0 = m.astype(jnp.bfloat16)
    r1 = m - m0.astype(jnp.float32)
    m1 = r1.astype(jnp.bfloat16)
    r2 = r1 - m1.astype(jnp.float32)
    m2 = r2.astype(jnp.bfloat16)
    return m0, m1, m2


def _hi16(m):
    u = jax.lax.bitcast_convert_type(m, jnp.uint32)
    return jax.lax.bitcast_convert_type(u & jnp.uint32(0xFFFF0000), jnp.float32)


def _split3_xla(m):
    h0 = _hi16(m)
    m0 = h0.astype(jnp.bfloat16)
    r1 = jax.lax.optimization_barrier(m - h0)
    h1 = _hi16(r1)
    m1 = h1.astype(jnp.bfloat16)
    r2 = jax.lax.optimization_barrier(r1 - h1)
    m2 = r2.astype(jnp.bfloat16)
    return m0, m1, m2


def _dot6(b, a0, a1, a2):
    b0, b1, b2 = _split3(b)
    acc = jnp.dot(b0, a0, preferred_element_type=jnp.float32)
    acc = acc + jnp.dot(b0, a1, preferred_element_type=jnp.float32)
    acc = acc + jnp.dot(b1, a0, preferred_element_type=jnp.float32)
    acc = acc + jnp.dot(b0, a2, preferred_element_type=jnp.float32)
    acc = acc + jnp.dot(b1, a1, preferred_element_type=jnp.float32)
    acc = acc + jnp.dot(b2, a0, preferred_element_type=jnp.float32)
    return acc


def _main_body(x_ref, wn_ref,
               a10_ref, a11_ref, a12_ref, a20_ref, a21_ref, a22_ref,
               proj_ref, v_ref):
    b = jax.lax.dot_general(x_ref[...], wn_ref[...], (((1,), (1,)), ((), ())),
                            preferred_element_type=jnp.float32)
    t = _dot6(b, a10_ref[...], a11_ref[...], a12_ref[...])
    v = _dot6(t, a20_ref[...], a21_ref[...], a22_ref[...])
    v_ref[...] = v
    proj_ref[...] = jnp.dot(v, wn_ref[...], preferred_element_type=jnp.float32)


def kernel(x, w, r):
    w_new = r @ w
    wn = w_new / jnp.linalg.norm(w_new, axis=1, keepdims=True)
    phi = wn @ wn.T

    lu, _, perm = jax.lax.linalg.lu(phi)
    eye = jnp.eye(K, dtype=jnp.float32)
    l_fac = jnp.tril(lu, -1) + eye
    u_fac = jnp.triu(lu)
    l_inv = jax.scipy.linalg.solve_triangular(l_fac, eye, lower=True,
                                              unit_diagonal=True)
    u_inv = jax.scipy.linalg.solve_triangular(u_fac, eye, lower=False)
    p_mat = jax.nn.one_hot(perm, K, dtype=jnp.float32)
    a1 = jnp.dot(l_inv, p_mat, precision=_HI).T
    a2 = u_inv.T
    a10, a11, a12 = _split3_xla(a1)
    a20, a21, a22 = _split3_xla(a2)

    proj, v = pl.pallas_call(
        _main_body,
        out_shape=[
            jax.ShapeDtypeStruct((B, N), jnp.float32),
            jax.ShapeDtypeStruct((B, K), jnp.float32),
        ],
        grid=(B // BM,),
        in_specs=[
            pl.BlockSpec((BM, N), lambda i: (i, 0)),
            pl.BlockSpec((K, N), lambda i: (0, 0)),
        ] + [pl.BlockSpec((K, K), lambda i: (0, 0))] * 6,
        out_specs=[
            pl.BlockSpec((BM, N), lambda i: (i, 0)),
            pl.BlockSpec((BM, K), lambda i: (i, 0)),
        ],
        compiler_params=pltpu.CompilerParams(
            dimension_semantics=("parallel",),
        ),
        name="spaceopt_main",
    )(x, wn, a10, a11, a12, a20, a21, a22)
    return (proj, v)


# speedup vs baseline: 1.2557x; 1.0514x over previous
import jax
import jax.numpy as jnp
from jax.experimental import pallas as pl
from jax.experimental.pallas import tpu as pltpu

B, N, K = 131072, 1024, 64
BM = 2048

_HI = jax.lax.Precision.HIGHEST


def _split3(m):
    m0 = m.astype(jnp.bfloat16)
    r1 = m - m0.astype(jnp.float32)
    m1 = r1.astype(jnp.bfloat16)
    r2 = r1 - m1.astype(jnp.float32)
    m2 = r2.astype(jnp.bfloat16)
    return m0, m1, m2


def _hi16(m):
    u = jax.lax.bitcast_convert_type(m, jnp.uint32)
    return jax.lax.bitcast_convert_type(u & jnp.uint32(0xFFFF0000), jnp.float32)


def _split3_xla(m):
    h0 = _hi16(m)
    m0 = h0.astype(jnp.bfloat16)
    r1 = jax.lax.optimization_barrier(m - h0)
    h1 = _hi16(r1)
    m1 = h1.astype(jnp.bfloat16)
    r2 = jax.lax.optimization_barrier(r1 - h1)
    m2 = r2.astype(jnp.bfloat16)
    return m0, m1, m2


def _dot6(b, a0, a1, a2):
    b0, b1, b2 = _split3(b)
    acc = jnp.dot(b0, a0, preferred_element_type=jnp.float32)
    acc = acc + jnp.dot(b0, a1, preferred_element_type=jnp.float32)
    acc = acc + jnp.dot(b1, a0, preferred_element_type=jnp.float32)
    acc = acc + jnp.dot(b0, a2, preferred_element_type=jnp.float32)
    acc = acc + jnp.dot(b1, a1, preferred_element_type=jnp.float32)
    acc = acc + jnp.dot(b2, a0, preferred_element_type=jnp.float32)
    return acc


def _main_body(x_ref, wnt_ref, wn_ref,
               a10_ref, a11_ref, a12_ref, a20_ref, a21_ref, a22_ref,
               proj_ref, v_ref):
    b = jnp.dot(x_ref[...], wnt_ref[...], preferred_element_type=jnp.float32)
    t = _dot6(b, a10_ref[...], a11_ref[...], a12_ref[...])
    v = _dot6(t, a20_ref[...], a21_ref[...], a22_ref[...])
    v_ref[...] = v
    proj_ref[...] = jnp.dot(v, wn_ref[...], preferred_element_type=jnp.float32)


def kernel(x, w, r):
    w_new = r @ w
    wn = w_new / jnp.linalg.norm(w_new, axis=1, keepdims=True)
    phi = wn @ wn.T

    lu, _, perm = jax.lax.linalg.lu(phi)
    eye = jnp.eye(K, dtype=jnp.float32)
    l_fac = jnp.tril(lu, -1) + eye
    u_fac = jnp.triu(lu)
    l_inv = jax.scipy.linalg.solve_triangular(l_fac, eye, lower=True,
                                              unit_diagonal=True)
    u_inv = jax.scipy.linalg.solve_triangular(u_fac, eye, lower=False)
    p_mat = jax.nn.one_hot(perm, K, dtype=jnp.float32)
    a1 = jnp.dot(l_inv, p_mat, precision=_HI).T
    a2 = u_inv.T
    a10, a11, a12 = _split3_xla(a1)
    a20, a21, a22 = _split3_xla(a2)
    wn_t = wn.T

    proj, v = pl.pallas_call(
        _main_body,
        out_shape=[
            jax.ShapeDtypeStruct((B, N), jnp.float32),
            jax.ShapeDtypeStruct((B, K), jnp.float32),
        ],
        grid=(B // BM,),
        in_specs=[
            pl.BlockSpec((BM, N), lambda i: (i, 0)),
            pl.BlockSpec((N, K), lambda i: (0, 0)),
            pl.BlockSpec((K, N), lambda i: (0, 0)),
        ] + [pl.BlockSpec((K, K), lambda i: (0, 0))] * 6,
        out_specs=[
            pl.BlockSpec((BM, N), lambda i: (i, 0)),
            pl.BlockSpec((BM, K), lambda i: (i, 0)),
        ],
        compiler_params=pltpu.CompilerParams(
            dimension_semantics=("parallel",),
        ),
        name="spaceopt_main",
    )(x, wn_t, wn, a10, a11, a12, a20, a21, a22)
    return (proj, v)


# speedup vs baseline: 1.4908x; 1.1873x over previous
import jax
import jax.numpy as jnp
from jax.experimental import pallas as pl
from jax.experimental.pallas import tpu as pltpu

B, N, K = 131072, 1024, 64
BM = 2048

_HI = jax.lax.Precision.HIGHEST


def _split3(m):
    m0 = m.astype(jnp.bfloat16)
    r1 = m - m0.astype(jnp.float32)
    m1 = r1.astype(jnp.bfloat16)
    r2 = r1 - m1.astype(jnp.float32)
    m2 = r2.astype(jnp.bfloat16)
    return m0, m1, m2


def _hi16(m):
    u = jax.lax.bitcast_convert_type(m, jnp.uint32)
    return jax.lax.bitcast_convert_type(u & jnp.uint32(0xFFFF0000), jnp.float32)


def _split3_xla(m):
    h0 = _hi16(m)
    m0 = h0.astype(jnp.bfloat16)
    r1 = jax.lax.optimization_barrier(m - h0)
    h1 = _hi16(r1)
    m1 = h1.astype(jnp.bfloat16)
    r2 = jax.lax.optimization_barrier(r1 - h1)
    m2 = r2.astype(jnp.bfloat16)
    return m0, m1, m2


def _dot6(b, a_stack):
    b0, b1, b2 = _split3(b)
    lhs = jnp.concatenate([b0, b0, b0, b1, b1, b2], axis=1)
    return jnp.dot(lhs, a_stack, preferred_element_type=jnp.float32)


def _main_body(x_ref, wnt_ref, wn_ref, a1s_ref, a2s_ref, proj_ref, v_ref):
    b = jnp.dot(x_ref[...], wnt_ref[...], preferred_element_type=jnp.float32)
    t = _dot6(b, a1s_ref[...])
    v = _dot6(t, a2s_ref[...])
    v_ref[...] = v
    proj_ref[...] = jnp.dot(v, wn_ref[...], preferred_element_type=jnp.float32)


def kernel(x, w, r):
    w_new = r @ w
    wn = w_new / jnp.linalg.norm(w_new, axis=1, keepdims=True)
    phi = wn @ wn.T

    lu, _, perm = jax.lax.linalg.lu(phi)
    eye = jnp.eye(K, dtype=jnp.float32)
    l_fac = jnp.tril(lu, -1) + eye
    u_fac = jnp.triu(lu)
    l_inv = jax.scipy.linalg.solve_triangular(l_fac, eye, lower=True,
                                              unit_diagonal=True)
    u_inv = jax.scipy.linalg.solve_triangular(u_fac, eye, lower=False)
    p_mat = jax.nn.one_hot(perm, K, dtype=jnp.float32)
    a1 = jnp.dot(l_inv, p_mat, precision=_HI).T
    a2 = u_inv.T
    a10, a11, a12 = _split3_xla(a1)
    a20, a21, a22 = _split3_xla(a2)
    a1s = jnp.concatenate([a10, a11, a12, a10, a11, a10], axis=0)
    a2s = jnp.concatenate([a20, a21, a22, a20, a21, a20], axis=0)
    wn_t = wn.T

    proj, v = pl.pallas_call(
        _main_body,
        out_shape=[
            jax.ShapeDtypeStruct((B, N), jnp.float32),
            jax.ShapeDtypeStruct((B, K), jnp.float32),
        ],
        grid=(B // BM,),
        in_specs=[
            pl.BlockSpec((BM, N), lambda i: (i, 0)),
            pl.BlockSpec((N, K), lambda i: (0, 0)),
            pl.BlockSpec((K, N), lambda i: (0, 0)),
        ] + [pl.BlockSpec((6 * K, K), lambda i: (0, 0))] * 2,
        out_specs=[
            pl.BlockSpec((BM, N), lambda i: (i, 0)),
            pl.BlockSpec((BM, K), lambda i: (i, 0)),
        ],
        compiler_params=pltpu.CompilerParams(
            dimension_semantics=("parallel",),
        ),
        name="spaceopt_main",
    )(x, wn_t, wn, a1s, a2s)
    return (proj, v)
